# Initial kernel scaffold; baseline (speedup 1.0000x reference)
import jax
import jax.numpy as jnp
from jax.experimental import pallas as pl


def kernel(node_feature, edge_feature, edge_src, edge_dst, W0, b0, We, be, Wn, bn, eps):
    raise NotImplementedError("write your pallas kernel here")


# speedup vs baseline: 1.9367x; 1.9367x over previous
import functools

import jax
import jax.numpy as jnp
from jax import lax
from jax.experimental import pallas as pl
from jax.experimental.pallas import tpu as pltpu
from jax.experimental.pallas import tpu_sc as plsc

N = 10000
E = 320000
D = 128
D_EDGE = 16
U = 128
STEPS = 4

NC = 2
NS = 16
LANES = 16

CHUNK = 128
KSUB = CHUNK // 128
NG = E // CHUNK
KMAX = (NG + 2 * NS - 1) // (2 * NS)
STAGE_TILES = 10
STAGE_ROWS = N // STAGE_TILES


def _proj0_body(nf_ref, w_ref, b_ref, out_ref):
    r = jnp.dot(nf_ref[...], w_ref[...], preferred_element_type=jnp.float32)
    out_ref[...] = r + b_ref[...]


def _proj0(node_feature, w0, b0):
    nb = 10
    bm = N // nb
    return pl.pallas_call(
        _proj0_body,
        grid=(nb,),
        in_specs=[
            pl.BlockSpec((bm, D), lambda i: (i, 0)),
            pl.BlockSpec((D, U), lambda i: (0, 0)),
            pl.BlockSpec((1, U), lambda i: (0, 0)),
        ],
        out_specs=pl.BlockSpec((bm, U), lambda i: (i, 0)),
        out_shape=jax.ShapeDtypeStruct((N, U), jnp.float32),
    )(node_feature, w0, b0.reshape(1, U))


def _eproj_body(ef_ref, w_ref, b_ref, *out_refs):
    r = jnp.dot(ef_ref[...], w_ref[...], preferred_element_type=jnp.float32)
    r = r + b_ref[...]
    for s in range(STEPS):
        out_refs[s][...] = r[:, s * U:(s + 1) * U]


def _eproj(edge_feature, we, be):
    wcat = jnp.transpose(we, (1, 0, 2)).reshape(D_EDGE, STEPS * U)
    bcat = be.reshape(1, STEPS * U)
    nb = 40
    bm = E // nb
    return pl.pallas_call(
        _eproj_body,
        grid=(nb,),
        in_specs=[
            pl.BlockSpec((bm, D_EDGE), lambda i: (i, 0)),
            pl.BlockSpec((D_EDGE, STEPS * U), lambda i: (0, 0)),
            pl.BlockSpec((1, STEPS * U), lambda i: (0, 0)),
        ],
        out_specs=[pl.BlockSpec((bm, U), lambda i: (i, 0))] * STEPS,
        out_shape=[jax.ShapeDtypeStruct((E, U), jnp.float32)] * STEPS,
    )(edge_feature, wcat, bcat)


def _update_body(x_ref, agg_ref, w_ref, b_ref, eps_ref, out_ref):
    h = eps_ref[0, 0] * x_ref[...] + agg_ref[0] + agg_ref[1]
    r = jnp.dot(h, w_ref[...], preferred_element_type=jnp.float32)
    out_ref[...] = r + b_ref[...]


def _update(x, agg, wn, bn, eps1):
    nb = 10
    bm = N // nb
    return pl.pallas_call(
        _update_body,
        grid=(nb,),
        in_specs=[
            pl.BlockSpec((bm, U), lambda i: (i, 0)),
            pl.BlockSpec((2, bm, U), lambda i: (0, i, 0)),
            pl.BlockSpec((U, U), lambda i: (0, 0)),
            pl.BlockSpec((1, U), lambda i: (0, 0)),
            pl.BlockSpec((1, 1), lambda i: (0, 0)),
        ],
        out_specs=pl.BlockSpec((bm, U), lambda i: (i, 0)),
        out_shape=jax.ShapeDtypeStruct((N, U), jnp.float32),
    )(x, agg, wn, bn.reshape(1, U), eps1.reshape(1, 1))


def _sc_body(x_hbm, ep_hbm, src_hbm, dst_hbm, zeros_hbm, agg_hbm,
             aggh, src_v, dst_v, ep_v, gx_v, sem):
    c = lax.axis_index("c")
    s = lax.axis_index("s")
    rows = pl.ds(s * STAGE_ROWS, STAGE_ROWS)

    @pl.when(s < STAGE_TILES)
    def _():
        pltpu.sync_copy(zeros_hbm.at[rows], aggh.at[rows])
    plsc.subcore_barrier()

    def chunk_body(k, carry):
        g = k * (2 * NS) + s * 2 + c

        @pl.when(g < NG)
        def _():
            pltpu.sync_copy(src_hbm.at[pl.ds(g * KSUB, KSUB)], src_v)
            pltpu.sync_copy(dst_hbm.at[pl.ds(g * KSUB, KSUB)], dst_v)
            pltpu.sync_copy(ep_hbm.at[pl.ds(g * CHUNK, CHUNK)], ep_v)
            descs = [
                pltpu.async_copy(x_hbm.at[src_v.at[j]],
                                 gx_v.at[pl.ds(j * 128, 128)], sem)
                for j in range(KSUB)
            ]
            for d in descs:
                d.wait()

            def row_body(r, rcarry):
                for j in range(U // LANES):
                    sl = pl.ds(j * LANES, LANES)
                    gx_v[r, sl] = jnp.maximum(gx_v[r, sl] + ep_v[r, sl], 0.0)
                return rcarry
            lax.fori_loop(0, CHUNK, row_body, 0, unroll=2)

            for j in range(KSUB):
                pltpu.sync_copy(gx_v.at[pl.ds(j * 128, 128)],
                                aggh.at[dst_v.at[j]], add=True)

        return carry

    lax.fori_loop(0, KMAX, chunk_body, 0)
    plsc.subcore_barrier()

    @pl.when(s < STAGE_TILES)
    def _():
        pltpu.sync_copy(aggh.at[rows], agg_hbm.at[c, rows])


@functools.lru_cache(maxsize=1)
def _sc_step():
  return pl.kernel(
    _sc_body,
    out_type=jax.ShapeDtypeStruct((2, N, U), jnp.float32),
    mesh=plsc.VectorSubcoreMesh(core_axis_name="c", subcore_axis_name="s",
                                num_cores=NC, num_subcores=NS),
    scratch_types=[
        pltpu.VMEM_SHARED((N, U), jnp.float32),
        pltpu.VMEM((KSUB, 128), jnp.int32),
        pltpu.VMEM((KSUB, 128), jnp.int32),
        pltpu.VMEM((CHUNK, U), jnp.float32),
        pltpu.VMEM((CHUNK, U), jnp.float32),
        pltpu.SemaphoreType.DMA,
    ],
  )


def kernel(node_feature, edge_feature, edge_src, edge_dst, W0, b0, We, be,
           Wn, bn, eps):
    eps = eps.astype(jnp.float32)
    eps_all = _eproj(edge_feature, We, be)
    x = _proj0(node_feature, W0, b0)
    zeros = jnp.zeros((N, U), jnp.float32)
    src2d = edge_src.reshape(E // 128, 128)
    dst2d = edge_dst.reshape(E // 128, 128)
    sc = _sc_step()
    feats = [x]
    for i in range(STEPS):
        agg = sc(x, eps_all[i], src2d, dst2d, zeros)
        x = _update(x, agg, Wn[i], bn[i], 1.0 + eps[i])
        feats.append(x)
    return jnp.stack(feats, axis=-2)


# speedup vs baseline: 2.3531x; 1.2150x over previous
import functools

import jax
import jax.numpy as jnp
from jax import lax
from jax.experimental import pallas as pl
from jax.experimental.pallas import tpu as pltpu
from jax.experimental.pallas import tpu_sc as plsc

N = 10000
E = 320000
D = 128
D_EDGE = 16
U = 128
STEPS = 4

NC = 2
NS = 16
LANES = 16

PAIR = 128
NP = E // PAIR
KP = (NP + 2 * NS - 1) // (2 * NS)
STAGE_TILES = 10
STAGE_ROWS = N // STAGE_TILES


def _proj0_body(nf_ref, w_ref, b_ref, out_ref):
    r = jnp.dot(nf_ref[...], w_ref[...], preferred_element_type=jnp.float32)
    out_ref[...] = r + b_ref[...]


def _proj0(node_feature, w0, b0):
    nb = 10
    bm = N // nb
    return pl.pallas_call(
        _proj0_body,
        grid=(nb,),
        in_specs=[
            pl.BlockSpec((bm, D), lambda i: (i, 0)),
            pl.BlockSpec((D, U), lambda i: (0, 0)),
            pl.BlockSpec((1, U), lambda i: (0, 0)),
        ],
        out_specs=pl.BlockSpec((bm, U), lambda i: (i, 0)),
        out_shape=jax.ShapeDtypeStruct((N, U), jnp.float32),
    )(node_feature, w0, b0.reshape(1, U))


def _eproj_body(ef_ref, w_ref, b_ref, *out_refs):
    r = jnp.dot(ef_ref[...], w_ref[...], preferred_element_type=jnp.float32)
    r = r + b_ref[...]
    for s in range(STEPS):
        out_refs[s][...] = r[:, s * U:(s + 1) * U]


def _eproj(edge_feature, we, be):
    wcat = jnp.transpose(we, (1, 0, 2)).reshape(D_EDGE, STEPS * U)
    bcat = be.reshape(1, STEPS * U)
    nb = 40
    bm = E // nb
    return pl.pallas_call(
        _eproj_body,
        grid=(nb,),
        in_specs=[
            pl.BlockSpec((bm, D_EDGE), lambda i: (i, 0)),
            pl.BlockSpec((D_EDGE, STEPS * U), lambda i: (0, 0)),
            pl.BlockSpec((1, STEPS * U), lambda i: (0, 0)),
        ],
        out_specs=[pl.BlockSpec((bm, U), lambda i: (i, 0))] * STEPS,
        out_shape=[jax.ShapeDtypeStruct((E, U), jnp.float32)] * STEPS,
    )(edge_feature, wcat, bcat)


def _update_body(x_ref, agg_ref, w_ref, b_ref, eps_ref, out_ref):
    h = eps_ref[0, 0] * x_ref[...] + agg_ref[0] + agg_ref[1]
    r = jnp.dot(h, w_ref[...], preferred_element_type=jnp.float32)
    out_ref[...] = r + b_ref[...]


def _update(x, agg, wn, bn, eps1):
    nb = 10
    bm = N // nb
    return pl.pallas_call(
        _update_body,
        grid=(nb,),
        in_specs=[
            pl.BlockSpec((bm, U), lambda i: (i, 0)),
            pl.BlockSpec((2, bm, U), lambda i: (0, i, 0)),
            pl.BlockSpec((U, U), lambda i: (0, 0)),
            pl.BlockSpec((1, U), lambda i: (0, 0)),
            pl.BlockSpec((1, 1), lambda i: (0, 0)),
        ],
        out_specs=pl.BlockSpec((bm, U), lambda i: (i, 0)),
        out_shape=jax.ShapeDtypeStruct((N, U), jnp.float32),
    )(x, agg, wn, bn.reshape(1, U), eps1.reshape(1, 1))


def _compute_half(ep_v, gx_v, h):
    def row_body(r, rcarry):
        for q in range(U // LANES):
            sl = pl.ds(q * LANES, LANES)
            gx_v[r, sl] = jnp.maximum(gx_v[r, sl] + ep_v[r, sl], 0.0)
        return rcarry
    lax.fori_loop(h * 64, (h + 1) * 64, row_body, 0, unroll=2)


def _sc_body(x_hbm, ep_hbm, src_hbm, dst_hbm, zeros_hbm, agg_hbm,
             aggh, src_v, dst_v, ep_v, gx_v,
             sem_i, sem_i2, sem_e, sem_g, sem_g2, sem_s, sem_s2):
    c = lax.axis_index("c")
    s = lax.axis_index("s")
    rows = pl.ds(s * STAGE_ROWS, STAGE_ROWS)

    @pl.when(s < STAGE_TILES)
    def _():
        pltpu.sync_copy(zeros_hbm.at[rows], aggh.at[rows])
    plsc.subcore_barrier()

    def pair_body(k, carry):
        p = k * (2 * NS) + s * 2 + c

        @pl.when(p < NP)
        def _():
            di = pltpu.async_copy(src_hbm.at[pl.ds(p * 2, 2)], src_v, sem_i)
            di2 = pltpu.async_copy(dst_hbm.at[pl.ds(p * 2, 2)], dst_v, sem_i2)
            de = pltpu.async_copy(ep_hbm.at[pl.ds(p * PAIR, PAIR)], ep_v,
                                  sem_e)
            di.wait()
            dg = pltpu.async_copy(x_hbm.at[src_v.at[0]],
                                  gx_v.at[pl.ds(0, 64)], sem_g)
            dg2 = pltpu.async_copy(x_hbm.at[src_v.at[1]],
                                   gx_v.at[pl.ds(64, 64)], sem_g2)
            de.wait()
            dg.wait()
            _compute_half(ep_v, gx_v, 0)
            di2.wait()
            dsc = pltpu.async_copy(gx_v.at[pl.ds(0, 64)],
                                   aggh.at[dst_v.at[0]], sem_s, add=True)
            dg2.wait()
            _compute_half(ep_v, gx_v, 1)
            dsc2 = pltpu.async_copy(gx_v.at[pl.ds(64, 64)],
                                    aggh.at[dst_v.at[1]], sem_s2, add=True)
            dsc.wait()
            dsc2.wait()

        return carry

    lax.fori_loop(0, KP, pair_body, 0)
    plsc.subcore_barrier()

    @pl.when(s < STAGE_TILES)
    def _():
        pltpu.sync_copy(aggh.at[rows], agg_hbm.at[c, rows])


@functools.lru_cache(maxsize=1)
def _sc_step():
  return pl.kernel(
    _sc_body,
    out_type=jax.ShapeDtypeStruct((2, N, U), jnp.float32),
    mesh=plsc.VectorSubcoreMesh(core_axis_name="c", subcore_axis_name="s",
                                num_cores=NC, num_subcores=NS),
    scratch_types=[
        pltpu.VMEM_SHARED((N, U), jnp.float32),
        pltpu.VMEM((2, 64), jnp.int32),
        pltpu.VMEM((2, 64), jnp.int32),
        pltpu.VMEM((PAIR, U), jnp.float32),
        pltpu.VMEM((PAIR, U), jnp.float32),
        pltpu.SemaphoreType.DMA,
        pltpu.SemaphoreType.DMA,
        pltpu.SemaphoreType.DMA,
        pltpu.SemaphoreType.DMA,
        pltpu.SemaphoreType.DMA,
        pltpu.SemaphoreType.DMA,
        pltpu.SemaphoreType.DMA,
    ],
  )


def kernel(node_feature, edge_feature, edge_src, edge_dst, W0, b0, We, be,
           Wn, bn, eps):
    eps = eps.astype(jnp.float32)
    eps_all = _eproj(edge_feature, We, be)
    x = _proj0(node_feature, W0, b0)
    zeros = jnp.zeros((N, U), jnp.float32)
    src2d = edge_src.reshape(E // 64, 64)
    dst2d = edge_dst.reshape(E // 64, 64)
    sc = _sc_step()
    feats = [x]
    for i in range(STEPS):
        agg = sc(x, eps_all[i], src2d, dst2d, zeros)
        x = _update(x, agg, Wn[i], bn[i], 1.0 + eps[i])
        feats.append(x)
    return jnp.stack(feats, axis=-2)


# speedup vs baseline: 4.2630x; 1.8116x over previous
import functools

import jax
import jax.numpy as jnp
from jax import lax
from jax.experimental import pallas as pl
from jax.experimental.pallas import tpu as pltpu
from jax.experimental.pallas import tpu_sc as plsc

N = 10000
E = 320000
D = 128
D_EDGE = 16
U = 128
STEPS = 4

NC = 2
NS = 16
LANES = 16

PAIR = 128
NP = E // PAIR
KP = (NP + 2 * NS - 1) // (2 * NS)
STAGE_TILES = 10
STAGE_ROWS = N // STAGE_TILES


def _proj0_body(nf_ref, w_ref, b_ref, out_ref):
    r = jnp.dot(nf_ref[...], w_ref[...], preferred_element_type=jnp.float32)
    out_ref[...] = r + b_ref[...]


def _proj0(node_feature, w0, b0):
    nb = 10
    bm = N // nb
    return pl.pallas_call(
        _proj0_body,
        grid=(nb,),
        in_specs=[
            pl.BlockSpec((bm, D), lambda i: (i, 0)),
            pl.BlockSpec((D, U), lambda i: (0, 0)),
            pl.BlockSpec((1, U), lambda i: (0, 0)),
        ],
        out_specs=pl.BlockSpec((bm, U), lambda i: (i, 0)),
        out_shape=jax.ShapeDtypeStruct((N, U), jnp.float32),
    )(node_feature, w0, b0.reshape(1, U))


def _eproj_body(ef_ref, w_ref, b_ref, *out_refs):
    r = jnp.dot(ef_ref[...], w_ref[...], preferred_element_type=jnp.float32)
    r = r + b_ref[...]
    for s in range(STEPS):
        out_refs[s][...] = r[:, s * U:(s + 1) * U]


def _eproj(edge_feature, we, be):
    wcat = jnp.transpose(we, (1, 0, 2)).reshape(D_EDGE, STEPS * U)
    bcat = be.reshape(1, STEPS * U)
    nb = 40
    bm = E // nb
    return pl.pallas_call(
        _eproj_body,
        grid=(nb,),
        in_specs=[
            pl.BlockSpec((bm, D_EDGE), lambda i: (i, 0)),
            pl.BlockSpec((D_EDGE, STEPS * U), lambda i: (0, 0)),
            pl.BlockSpec((1, STEPS * U), lambda i: (0, 0)),
        ],
        out_specs=[pl.BlockSpec((bm, U), lambda i: (i, 0))] * STEPS,
        out_shape=[jax.ShapeDtypeStruct((E, U), jnp.float32)] * STEPS,
    )(edge_feature, wcat, bcat)


def _update_body(x_ref, agg_ref, w_ref, b_ref, eps_ref, out_ref):
    h = eps_ref[0, 0] * x_ref[...] + agg_ref[0] + agg_ref[1]
    r = jnp.dot(h, w_ref[...], preferred_element_type=jnp.float32)
    out_ref[...] = r + b_ref[...]


def _update(x, agg, wn, bn, eps1):
    nb = 10
    bm = N // nb
    return pl.pallas_call(
        _update_body,
        grid=(nb,),
        in_specs=[
            pl.BlockSpec((bm, U), lambda i: (i, 0)),
            pl.BlockSpec((2, bm, U), lambda i: (0, i, 0)),
            pl.BlockSpec((U, U), lambda i: (0, 0)),
            pl.BlockSpec((1, U), lambda i: (0, 0)),
            pl.BlockSpec((1, 1), lambda i: (0, 0)),
        ],
        out_specs=pl.BlockSpec((bm, U), lambda i: (i, 0)),
        out_shape=jax.ShapeDtypeStruct((N, U), jnp.float32),
    )(x, agg, wn, bn.reshape(1, U), eps1.reshape(1, 1))


def _compute_half(ep_v, gx_v, h):
    @plsc.parallel_loop(h * 64, (h + 1) * 64, step=1, unroll=4)
    def _(r):
        for q in range(U // LANES):
            sl = pl.ds(q * LANES, LANES)
            gx_v[r, sl] = jnp.maximum(gx_v[r, sl] + ep_v[r, sl], 0.0)


def _sc_body(x_hbm, ep_hbm, src_hbm, dst_hbm, zeros_hbm, agg_hbm,
             aggh, src_v, dst_v, ep_v, gx_v,
             sem_i, sem_i2, sem_e, sem_g, sem_g2, sem_s, sem_s2):
    c = lax.axis_index("c")
    s = lax.axis_index("s")
    rows = pl.ds(s * STAGE_ROWS, STAGE_ROWS)

    @pl.when(s < STAGE_TILES)
    def _():
        pltpu.sync_copy(zeros_hbm.at[rows], aggh.at[rows])
    plsc.subcore_barrier()

    def pair_body(k, carry):
        p = k * (2 * NS) + s * 2 + c

        @pl.when(p < NP)
        def _():
            di = pltpu.async_copy(src_hbm.at[pl.ds(p * 2, 2)], src_v, sem_i)
            di2 = pltpu.async_copy(dst_hbm.at[pl.ds(p * 2, 2)], dst_v, sem_i2)
            de = pltpu.async_copy(ep_hbm.at[pl.ds(p * PAIR, PAIR)], ep_v,
                                  sem_e)
            di.wait()
            dg = pltpu.async_copy(x_hbm.at[src_v.at[0]],
                                  gx_v.at[pl.ds(0, 64)], sem_g)
            dg2 = pltpu.async_copy(x_hbm.at[src_v.at[1]],
                                   gx_v.at[pl.ds(64, 64)], sem_g2)
            de.wait()
            dg.wait()
            _compute_half(ep_v, gx_v, 0)
            di2.wait()
            dsc = pltpu.async_copy(gx_v.at[pl.ds(0, 64)],
                                   aggh.at[dst_v.at[0]], sem_s, add=True)
            dg2.wait()
            _compute_half(ep_v, gx_v, 1)
            dsc2 = pltpu.async_copy(gx_v.at[pl.ds(64, 64)],
                                    aggh.at[dst_v.at[1]], sem_s2, add=True)
            dsc.wait()
            dsc2.wait()

        return carry

    lax.fori_loop(0, KP, pair_body, 0)
    plsc.subcore_barrier()

    @pl.when(s < STAGE_TILES)
    def _():
        pltpu.sync_copy(aggh.at[rows], agg_hbm.at[c, rows])


@functools.lru_cache(maxsize=1)
def _sc_step():
  return pl.kernel(
    _sc_body,
    out_type=jax.ShapeDtypeStruct((2, N, U), jnp.float32),
    mesh=plsc.VectorSubcoreMesh(core_axis_name="c", subcore_axis_name="s",
                                num_cores=NC, num_subcores=NS),
    scratch_types=[
        pltpu.VMEM_SHARED((N, U), jnp.float32),
        pltpu.VMEM((2, 64), jnp.int32),
        pltpu.VMEM((2, 64), jnp.int32),
        pltpu.VMEM((PAIR, U), jnp.float32),
        pltpu.VMEM((PAIR, U), jnp.float32),
        pltpu.SemaphoreType.DMA,
        pltpu.SemaphoreType.DMA,
        pltpu.SemaphoreType.DMA,
        pltpu.SemaphoreType.DMA,
        pltpu.SemaphoreType.DMA,
        pltpu.SemaphoreType.DMA,
        pltpu.SemaphoreType.DMA,
    ],
  )


def kernel(node_feature, edge_feature, edge_src, edge_dst, W0, b0, We, be,
           Wn, bn, eps):
    eps = eps.astype(jnp.float32)
    eps_all = _eproj(edge_feature, We, be)
    x = _proj0(node_feature, W0, b0)
    zeros = jnp.zeros((N, U), jnp.float32)
    src2d = edge_src.reshape(E // 64, 64)
    dst2d = edge_dst.reshape(E // 64, 64)
    sc = _sc_step()
    feats = [x]
    for i in range(STEPS):
        agg = sc(x, eps_all[i], src2d, dst2d, zeros)
        x = _update(x, agg, Wn[i], bn[i], 1.0 + eps[i])
        feats.append(x)
    return jnp.stack(feats, axis=-2)
